# baseline (speedup 1.0000x reference)
import numpy as np
import jax
import jax.numpy as jnp
from jax.experimental import pallas as pl

_ATOMS = 51
_V_MIN = -10.0
_V_MAX = 10.0
_LANES = 128
_WIDTH = _ATOMS * _LANES


def _projection_matrix():
    atoms = _ATOMS
    delta = np.float32((_V_MAX - _V_MIN) / (atoms - 1))
    supports = np.linspace(_V_MIN, _V_MAX, atoms, dtype=np.float32)
    tz = np.clip(supports, np.float32(_V_MIN), np.float32(_V_MAX))
    b = ((tz - np.float32(_V_MIN)) / delta).astype(np.float32)
    l = np.floor(b).astype(np.int32)
    u = np.ceil(b).astype(np.int32)
    l = np.where((u > 0) & (l == u), l - 1, l)
    u = np.where((l < atoms - 1) & (l == u), u + 1, u)
    wl = (u.astype(np.float32) - b).astype(np.float32)
    wu = (b - l.astype(np.float32)).astype(np.float32)
    cols = np.arange(atoms)
    mc = np.zeros((atoms, atoms), np.float32)
    np.add.at(mc, (l, cols), wl)
    np.add.at(mc, (u, cols), wu)
    return mc


def _flat_kernel(a_ref, f_ref, c_ref, o_ref):
    i = pl.program_id(0)
    g = jnp.log(f_ref[...] + 1e-16)
    gl = g * c_ref[0:1, :]
    gl += jnp.roll(g, 1, axis=1) * c_ref[1:2, :]
    gl += jnp.roll(g, -1, axis=1) * c_ref[2:3, :]
    part = jnp.sum(a_ref[...] * gl, keepdims=True)

    @pl.when(i == 0)
    def _init():
        o_ref[...] = jnp.zeros_like(o_ref)

    o_ref[...] += part[0:1, 0:1]


def _matmul_kernel(a_ref, f_ref, m_ref, o_ref):
    i = pl.program_id(0)
    g = jnp.log(f_ref[...] + 1e-16)
    gl = jnp.dot(g, m_ref[...], preferred_element_type=jnp.float32)
    part = jnp.sum(a_ref[...] * gl, keepdims=True)

    @pl.when(i == 0)
    def _init():
        o_ref[...] = jnp.zeros_like(o_ref)

    o_ref[...] += part


def kernel(anchor, feature):
    batch, atoms = anchor.shape
    mc_np = _projection_matrix()
    tridiag = np.array_equal(mc_np, np.tril(np.triu(mc_np, -1), 1))
    total_elems = batch * atoms

    if tridiag and atoms == _ATOMS and total_elems % _WIDTH == 0:
        rows = total_elems // _WIDTH
        c0 = np.tile(np.diag(mc_np), _LANES)
        cm = np.tile(np.concatenate([[0.0], np.diag(mc_np, 1)]), _LANES)
        cp = np.tile(np.concatenate([np.diag(mc_np, -1), [0.0]]), _LANES)
        coefs = np.zeros((8, _WIDTH), np.float32)
        coefs[0], coefs[1], coefs[2] = c0, cm, cp
        coefs = jnp.asarray(coefs)

        a2 = anchor.reshape(rows, _WIDTH)
        f2 = feature.reshape(rows, _WIDTH)
        blk = 128
        while rows % blk:
            blk //= 2
        grid = rows // blk
        total = pl.pallas_call(
            _flat_kernel,
            grid=(grid,),
            in_specs=[
                pl.BlockSpec((blk, _WIDTH), lambda i: (i, 0)),
                pl.BlockSpec((blk, _WIDTH), lambda i: (i, 0)),
                pl.BlockSpec((8, _WIDTH), lambda i: (0, 0)),
            ],
            out_specs=pl.BlockSpec((1, 1), lambda i: (0, 0)),
            out_shape=jax.ShapeDtypeStruct((1, 1), jnp.float32),
        )(a2, f2, coefs)
    else:
        blk = 4096
        while batch % blk:
            blk //= 2
        grid = batch // blk
        total = pl.pallas_call(
            _matmul_kernel,
            grid=(grid,),
            in_specs=[
                pl.BlockSpec((blk, atoms), lambda i: (i, 0)),
                pl.BlockSpec((blk, atoms), lambda i: (i, 0)),
                pl.BlockSpec((atoms, atoms), lambda i: (0, 0)),
            ],
            out_specs=pl.BlockSpec((1, 1), lambda i: (0, 0)),
            out_shape=jax.ShapeDtypeStruct((1, 1), jnp.float32),
        )(anchor, feature, jnp.asarray(mc_np))

    return -(total[0, 0] / batch)


# speedup vs baseline: 1.3471x; 1.3471x over previous
import numpy as np
import jax
import jax.numpy as jnp
from jax.experimental import pallas as pl

_ATOMS = 51
_V_MIN = -10.0
_V_MAX = 10.0
_LANES = 128
_WIDTH = _ATOMS * _LANES


def _projection_matrix():
    atoms = _ATOMS
    delta = np.float32((_V_MAX - _V_MIN) / (atoms - 1))
    supports = np.linspace(_V_MIN, _V_MAX, atoms, dtype=np.float32)
    tz = np.clip(supports, np.float32(_V_MIN), np.float32(_V_MAX))
    b = ((tz - np.float32(_V_MIN)) / delta).astype(np.float32)
    l = np.floor(b).astype(np.int32)
    u = np.ceil(b).astype(np.int32)
    l = np.where((u > 0) & (l == u), l - 1, l)
    u = np.where((l < atoms - 1) & (l == u), u + 1, u)
    wl = (u.astype(np.float32) - b).astype(np.float32)
    wu = (b - l.astype(np.float32)).astype(np.float32)
    cols = np.arange(atoms)
    mc = np.zeros((atoms, atoms), np.float32)
    np.add.at(mc, (l, cols), wl)
    np.add.at(mc, (u, cols), wu)
    return mc


def _flat_kernel(a_ref, f_ref, c_ref, o_ref):
    i = pl.program_id(0)
    g = jnp.log(f_ref[...] + 1e-16)
    gl = g * c_ref[0:1, :]
    gl += jnp.roll(g, 1, axis=1) * c_ref[1:2, :]
    gl += jnp.roll(g, -1, axis=1) * c_ref[2:3, :]
    part = jnp.sum(a_ref[...] * gl, keepdims=True)

    @pl.when(i == 0)
    def _init():
        o_ref[...] = jnp.zeros_like(o_ref)

    o_ref[...] += part[0:1, 0:1]


def _matmul_kernel(a_ref, f_ref, m_ref, o_ref):
    i = pl.program_id(0)
    g = jnp.log(f_ref[...] + 1e-16)
    gl = jnp.dot(g, m_ref[...], preferred_element_type=jnp.float32)
    part = jnp.sum(a_ref[...] * gl, keepdims=True)

    @pl.when(i == 0)
    def _init():
        o_ref[...] = jnp.zeros_like(o_ref)

    o_ref[...] += part


def kernel(anchor, feature):
    batch, atoms = anchor.shape
    mc_np = _projection_matrix()
    tridiag = np.array_equal(mc_np, np.tril(np.triu(mc_np, -1), 1))
    total_elems = batch * atoms

    if False and tridiag and atoms == _ATOMS and total_elems % _WIDTH == 0:
        rows = total_elems // _WIDTH
        c0 = np.tile(np.diag(mc_np), _LANES)
        cm = np.tile(np.concatenate([[0.0], np.diag(mc_np, 1)]), _LANES)
        cp = np.tile(np.concatenate([np.diag(mc_np, -1), [0.0]]), _LANES)
        coefs = np.zeros((8, _WIDTH), np.float32)
        coefs[0], coefs[1], coefs[2] = c0, cm, cp
        coefs = jnp.asarray(coefs)

        a2 = anchor.reshape(rows, _WIDTH)
        f2 = feature.reshape(rows, _WIDTH)
        blk = 128
        while rows % blk:
            blk //= 2
        grid = rows // blk
        total = pl.pallas_call(
            _flat_kernel,
            grid=(grid,),
            in_specs=[
                pl.BlockSpec((blk, _WIDTH), lambda i: (i, 0)),
                pl.BlockSpec((blk, _WIDTH), lambda i: (i, 0)),
                pl.BlockSpec((8, _WIDTH), lambda i: (0, 0)),
            ],
            out_specs=pl.BlockSpec((1, 1), lambda i: (0, 0)),
            out_shape=jax.ShapeDtypeStruct((1, 1), jnp.float32),
        )(a2, f2, coefs)
    else:
        blk = 16384
        while batch % blk:
            blk //= 2
        grid = batch // blk
        total = pl.pallas_call(
            _matmul_kernel,
            grid=(grid,),
            in_specs=[
                pl.BlockSpec((blk, atoms), lambda i: (i, 0)),
                pl.BlockSpec((blk, atoms), lambda i: (i, 0)),
                pl.BlockSpec((atoms, atoms), lambda i: (0, 0)),
            ],
            out_specs=pl.BlockSpec((1, 1), lambda i: (0, 0)),
            out_shape=jax.ShapeDtypeStruct((1, 1), jnp.float32),
        )(anchor, feature, jnp.asarray(mc_np))

    return -(total[0, 0] / batch)
